# baseline (speedup 1.0000x reference)
import jax
import jax.numpy as jnp
from jax import lax
from jax.experimental import pallas as pl
from jax.experimental.pallas import tpu as pltpu
from jax.experimental.pallas import tpu_sc as plsc

_N = 10000
_E = 320000
_D = 128

_NC = 2
_NS = 16
_NWK = _NC * _NS
_CH = 128
_NCHK = _E // _CH
_CREM = _NCHK % _NWK
_CBASE = _NCHK // _NWK
_PCAP = (_CBASE + 1) * _CH
_NP = 10240
_RPT = _NP // _NS
_BR = 2000
_CW = 16
_DUMP = _N + 64


def _build_ext_body(x_ref, f_ref, xz_ref, flag_ref):
    f = f_ref[...]
    xz_ref[...] = x_ref[...] * (1.0 - f)
    flag_ref[...] = jnp.max(f).reshape(1, 1)


def _sc_compact_body(src2d_hbm, dst2d_hbm, fr_hbm, csrc_hbm, cdst_hbm, nch_hbm,
                     fbuf, sbuf, dbuf, osrc, odst, cvec):
    cid = lax.axis_index("c")
    sid = lax.axis_index("s")
    wid = sid * _NC + cid
    c0 = wid * _CBASE + lax.min(wid, _CREM)
    cnt = _CBASE + jnp.where(wid < _CREM, 1, 0)

    pltpu.sync_copy(fr_hbm, fbuf)

    zs = jnp.zeros((16,), jnp.int32)
    dmp = jnp.full((16,), _DUMP, jnp.int32)

    def prefill(i, carry):
        osrc[pl.ds(i * 16, 16)] = zs
        odst[pl.ds(i * 16, 16)] = dmp
        return carry

    lax.fori_loop(0, _PCAP // 16, prefill, 0)

    def chunk(j, cur):
        pltpu.sync_copy(src2d_hbm.at[c0 + j], sbuf)
        pltpu.sync_copy(dst2d_hbm.at[c0 + j], dbuf)
        for t in range(_CH // 16):
            sv = sbuf[pl.ds(t * 16, 16)]
            dv = dbuf[pl.ds(t * 16, 16)]
            fv = plsc.load_gather(fbuf, [sv])
            m = fv == 0
            plsc.store_compressed(osrc.at[pl.ds(cur, 16)], sv, mask=m)
            plsc.store_compressed(odst.at[pl.ds(cur, 16)], dv, mask=m)
            cur = cur + jnp.max(plsc.all_reduce_population_count(m))
        return cur

    cur = lax.fori_loop(0, cnt, chunk, 0)
    nchw = (cur + _CH - 1) // _CH

    pltpu.sync_copy(osrc, csrc_hbm.at[wid])
    pltpu.sync_copy(odst, cdst_hbm.at[wid])
    cvec[...] = jnp.broadcast_to(nchw, (16,))
    pltpu.sync_copy(cvec, nch_hbm.at[wid])


def _sc_scatter_body(xz_hbm, dst2d_hbm, csrc_hbm, cdst_hbm, nch_hbm,
                     zeros_hbm, zeros2_hbm, ones_hbm, acc_out_hbm, cnt_out_hbm,
                     sidx, didx, rows, dida, didt, ones_v, cvec,
                     isems, gsems, ssems, isemsA, ssemsA, acc, acc2):
    cid = lax.axis_index("c")
    sid = lax.axis_index("s")
    wid = sid * _NC + cid
    r0 = sid * _RPT
    pltpu.sync_copy(zeros_hbm.at[pl.ds(r0, _RPT)], acc.at[pl.ds(r0, _RPT)])
    pltpu.sync_copy(zeros2_hbm.at[pl.ds(r0, _RPT)], acc2.at[pl.ds(r0, _RPT)])
    pltpu.sync_copy(ones_hbm, ones_v)
    pltpu.sync_copy(nch_hbm.at[wid], cvec)
    plsc.subcore_barrier()

    c0 = wid * _CBASE + lax.min(wid, _CREM)
    cnt = _CBASE + jnp.where(wid < _CREM, 1, 0)

    def groupA(i, carry):
        c = c0 + i * 8
        ia = pltpu.async_copy(dst2d_hbm.at[pl.ds(c, 4)], dida[0], isemsA[0])
        ib = pltpu.async_copy(dst2d_hbm.at[pl.ds(c + 4, 4)], dida[1], isemsA[1])
        ia.wait()
        sa = [pltpu.async_copy(ones_v, acc2.at[dida[0].at[b]], ssemsA[0],
                               add=True) for b in range(4)]
        ib.wait()
        sb = [pltpu.async_copy(ones_v, acc2.at[dida[1].at[b]], ssemsA[1],
                               add=True) for b in range(4)]
        for s in sa + sb:
            s.wait()
        return carry

    lax.fori_loop(0, cnt // 8, groupA, 0)

    def tailA(i, carry):
        c = c0 + (cnt // 8) * 8 + i
        pltpu.async_copy(dst2d_hbm.at[c], didt, isemsA[0]).wait()
        pltpu.async_copy(ones_v, acc2.at[didt], ssemsA[0], add=True).wait()
        return carry

    lax.fori_loop(0, cnt - (cnt // 8) * 8, tailA, 0)

    nchw = jnp.max(cvec[...])

    def groupB(i, carry):
        o0 = pl.multiple_of((i * 2) * _CH, 8)
        o1 = pl.multiple_of((i * 2 + 1) * _CH, 8)
        i0 = pltpu.async_copy(csrc_hbm.at[wid, pl.ds(o0, _CH)], sidx[0], isems[0])
        i0b = pltpu.async_copy(cdst_hbm.at[wid, pl.ds(o0, _CH)], didx[0], isems[0])
        i1 = pltpu.async_copy(csrc_hbm.at[wid, pl.ds(o1, _CH)], sidx[1], isems[1])
        i1b = pltpu.async_copy(cdst_hbm.at[wid, pl.ds(o1, _CH)], didx[1], isems[1])
        i0.wait(); i0b.wait()
        g0 = pltpu.async_copy(xz_hbm.at[sidx[0]], rows[0], gsems[0])
        i1.wait(); i1b.wait()
        g1 = pltpu.async_copy(xz_hbm.at[sidx[1]], rows[1], gsems[1])
        g0.wait()
        s0 = pltpu.async_copy(rows[0], acc.at[didx[0]], ssems[0], add=True)
        g1.wait()
        s1 = pltpu.async_copy(rows[1], acc.at[didx[1]], ssems[1], add=True)
        s0.wait(); s1.wait()
        return carry

    lax.fori_loop(0, nchw // 2, groupB, 0)

    def tailB(i, carry):
        o0 = pl.multiple_of(((nchw // 2) * 2 + i) * _CH, 8)
        ia = pltpu.async_copy(csrc_hbm.at[wid, pl.ds(o0, _CH)], sidx[0], isems[0])
        ib = pltpu.async_copy(cdst_hbm.at[wid, pl.ds(o0, _CH)], didx[0], isems[0])
        ia.wait(); ib.wait()
        pltpu.async_copy(xz_hbm.at[sidx[0]], rows[0], gsems[0]).wait()
        pltpu.async_copy(rows[0], acc.at[didx[0]], ssems[0], add=True).wait()
        return carry

    lax.fori_loop(0, nchw - (nchw // 2) * 2, tailB, 0)

    plsc.subcore_barrier()
    pltpu.sync_copy(acc.at[pl.ds(r0, _RPT)], acc_out_hbm.at[cid, pl.ds(r0, _RPT)])
    pltpu.sync_copy(acc2.at[pl.ds(r0, _RPT)], cnt_out_hbm.at[cid, pl.ds(r0, _RPT)])


def _combine_body(acc_ref, cnt_ref, x_ref, f_ref, agg_ref, wn_ref, b_ref,
                  wr_ref, flag_ref, out_ref):
    summed = acc_ref[0] + acc_ref[1]
    count = cnt_ref[0, :, 0:1] + cnt_ref[1, :, 0:1]
    mean = summed / jnp.maximum(count, 1.0)
    f = f_ref[...]
    xz = x_ref[...] * (1.0 - f)
    agg = agg_ref[...]
    use_hybrid = flag_ref[0, 0] > 0.0
    target = (jnp.sum(jnp.abs(agg), axis=1, keepdims=True) > 0.0) & use_hybrid
    neigh_in = jnp.where(target, agg, mean)
    root_in = jnp.where(target, 0.0, xz)
    out_ref[...] = (
        jnp.dot(neigh_in, wn_ref[...], preferred_element_type=jnp.float32)
        + b_ref[...]
        + jnp.dot(root_in, wr_ref[...], preferred_element_type=jnp.float32))


def kernel(x, edge_index, frontier_mask, aggregated_neighbors,
           W_neigh, b_neigh, W_root):
    f = frontier_mask.astype(jnp.float32).reshape(_N, 1)
    fr_i = frontier_mask.astype(jnp.int32)
    src2d = edge_index[0].reshape(_NCHK, _CH)
    dst2d = edge_index[1].reshape(_NCHK, _CH)
    zeros = jnp.zeros((_NP, _D), jnp.float32)
    zeros2 = jnp.zeros((_NP, _CW), jnp.float32)
    ones = jnp.ones((_CH, _CW), jnp.float32)
    b2 = b_neigh.reshape(1, _D)

    xz, flag = pl.pallas_call(
        _build_ext_body,
        out_shape=[jax.ShapeDtypeStruct((_N, _D), jnp.float32),
                   jax.ShapeDtypeStruct((1, 1), jnp.float32)],
    )(x, f)

    mesh = plsc.VectorSubcoreMesh(core_axis_name="c", subcore_axis_name="s")
    sc_compact = pl.kernel(
        _sc_compact_body,
        mesh=mesh,
        compiler_params=pltpu.CompilerParams(use_tc_tiling_on_sc=False, needs_layout_passes=False),
        out_type=[jax.ShapeDtypeStruct((_NWK, _PCAP), jnp.int32),
                  jax.ShapeDtypeStruct((_NWK, _PCAP), jnp.int32),
                  jax.ShapeDtypeStruct((_NWK, 16), jnp.int32)],
        scratch_types=[
            pltpu.VMEM((_N,), jnp.int32),
            pltpu.VMEM((_CH,), jnp.int32),
            pltpu.VMEM((_CH,), jnp.int32),
            pltpu.VMEM((_PCAP,), jnp.int32),
            pltpu.VMEM((_PCAP,), jnp.int32),
            pltpu.VMEM((16,), jnp.int32),
        ],
    )
    csrc, cdst, nch = sc_compact(src2d, dst2d, fr_i)

    sc_scatter = pl.kernel(
        _sc_scatter_body,
        mesh=mesh,
        compiler_params=pltpu.CompilerParams(use_tc_tiling_on_sc=False, needs_layout_passes=False),
        out_type=[jax.ShapeDtypeStruct((_NC, _NP, _D), jnp.float32),
                  jax.ShapeDtypeStruct((_NC, _NP, _CW), jnp.float32)],
        scratch_types=[
            [pltpu.VMEM((_CH,), jnp.int32) for _ in range(2)],
            [pltpu.VMEM((_CH,), jnp.int32) for _ in range(2)],
            [pltpu.VMEM((_CH, _D), jnp.float32) for _ in range(2)],
            [pltpu.VMEM((4, _CH), jnp.int32) for _ in range(2)],
            pltpu.VMEM((_CH,), jnp.int32),
            pltpu.VMEM((_CH, _CW), jnp.float32),
            pltpu.VMEM((16,), jnp.int32),
            [pltpu.SemaphoreType.DMA for _ in range(2)],
            [pltpu.SemaphoreType.DMA for _ in range(2)],
            [pltpu.SemaphoreType.DMA for _ in range(2)],
            [pltpu.SemaphoreType.DMA for _ in range(2)],
            [pltpu.SemaphoreType.DMA for _ in range(2)],
            pltpu.VMEM_SHARED((_NP, _D), jnp.float32),
            pltpu.VMEM_SHARED((_NP, _CW), jnp.float32),
        ],
    )
    acc, cnt = sc_scatter(xz, dst2d, csrc, cdst, nch, zeros, zeros2, ones)

    out = pl.pallas_call(
        _combine_body,
        grid=(_N // _BR,),
        in_specs=[
            pl.BlockSpec((_NC, _BR, _D), lambda i: (0, i, 0)),
            pl.BlockSpec((_NC, _BR, _CW), lambda i: (0, i, 0)),
            pl.BlockSpec((_BR, _D), lambda i: (i, 0)),
            pl.BlockSpec((_BR, 1), lambda i: (i, 0)),
            pl.BlockSpec((_BR, _D), lambda i: (i, 0)),
            pl.BlockSpec((_D, _D), lambda i: (0, 0)),
            pl.BlockSpec((1, _D), lambda i: (0, 0)),
            pl.BlockSpec((_D, _D), lambda i: (0, 0)),
            pl.BlockSpec((1, 1), lambda i: (0, 0)),
        ],
        out_specs=pl.BlockSpec((_BR, _D), lambda i: (i, 0)),
        out_shape=jax.ShapeDtypeStruct((_N, _D), jnp.float32),
    )(acc, cnt, x, f, aggregated_neighbors, W_neigh, b2, W_root, flag)
    return out


# speedup vs baseline: 1.0944x; 1.0944x over previous
import jax
import jax.numpy as jnp
from jax import lax
from jax.experimental import pallas as pl
from jax.experimental.pallas import tpu as pltpu
from jax.experimental.pallas import tpu_sc as plsc

_N = 10000
_E = 320000
_D = 128

_NC = 2
_NS = 16
_NWK = _NC * _NS
_CH = 128
_NCHK = _E // _CH
_CREM = _NCHK % _NWK
_CBASE = _NCHK // _NWK
_CMAX = _CBASE + 1
_BLK = 4
_PCAP = 80 * _CH
_NP = 10240
_RPT = _NP // _NS
_BR = 2000
_DUMP = _N + 64
_CROWS = _NP // 16


def _build_xz_body(x_ref, f_ref, xz_ref, flag_ref):
    f = f_ref[...]
    xz_ref[...] = x_ref[...] * (1.0 - f)
    flag_ref[...] = jnp.max(f).reshape(1, 1)


def _sc_compact_body(src2d_hbm, dst2d_hbm, fr_hbm, iota_hbm, csrc_hbm,
                     cdst_hbm, nch_hbm, cnt_out_hbm, fbuf, sall, dall, osrc,
                     odst, cbuf, iotab, cvec, cnt2):
    cid = lax.axis_index("c")
    sid = lax.axis_index("s")
    wid = sid * _NC + cid
    c0 = wid * _CBASE + lax.min(wid, _CREM)
    cnt = _CBASE + jnp.where(wid < _CREM, 1, 0)
    r0c = sid * (_CROWS // _NS)

    pltpu.sync_copy(fr_hbm, fbuf)
    pltpu.sync_copy(iota_hbm, iotab)
    pltpu.sync_copy(src2d_hbm.at[pl.ds(c0, _CMAX)], sall)
    pltpu.sync_copy(dst2d_hbm.at[pl.ds(c0, _CMAX)], dall)

    zsv = jnp.zeros((16,), jnp.float32)

    def zero_cbuf(i, carry):
        cbuf[i] = zsv
        return carry

    lax.fori_loop(0, _CROWS, zero_cbuf, 0)
    pltpu.sync_copy(cbuf.at[pl.ds(0, _CROWS // _NS)],
                    cnt2.at[pl.ds(r0c, _CROWS // _NS)])

    zs = jnp.zeros((16,), jnp.int32)
    dmp = jnp.full((16,), _DUMP, jnp.int32)

    def prefill(i, carry):
        osrc[pl.ds(i * 16, 16)] = zs
        odst[pl.ds(i * 16, 16)] = dmp
        return carry

    lax.fori_loop(0, _PCAP // 16, prefill, 0)
    plsc.subcore_barrier()

    onev = jnp.ones((16,), jnp.float32)

    def chunk(j, cur):
        for t in range(_CH // 16):
            sv = sall[j, pl.ds(t * 16, 16)]
            dv = dall[j, pl.ds(t * 16, 16)]
            plsc.addupdate_scatter(cbuf, [dv >> 4, dv & 15], onev)
            fv = plsc.load_gather(fbuf, [sv])
            m = fv == 0
            plsc.store_compressed(osrc.at[pl.ds(cur, 16)], sv, mask=m)
            plsc.store_compressed(odst.at[pl.ds(cur, 16)], dv, mask=m)
            cur = cur + jnp.max(plsc.all_reduce_population_count(m))
        return cur

    cur = lax.fori_loop(0, cnt, chunk, jnp.int32(0))
    nblk = (cur + _BLK * _CH - 1) // (_BLK * _CH)

    for k in range(_CROWS // _CH):
        pltpu.sync_copy(cbuf.at[pl.ds(k * _CH, _CH)], cnt2.at[iotab.at[k]],
                        add=True)

    pltpu.sync_copy(osrc, csrc_hbm.at[wid])
    pltpu.sync_copy(odst, cdst_hbm.at[wid])
    cvec[...] = jnp.broadcast_to(nblk, (16,))
    pltpu.sync_copy(cvec, nch_hbm.at[wid])

    plsc.subcore_barrier()
    pltpu.sync_copy(cnt2.at[pl.ds(r0c, _CROWS // _NS)],
                    cnt_out_hbm.at[cid, pl.ds(r0c, _CROWS // _NS)])


def _sc_scatter_body(xz_hbm, csrc_hbm, cdst_hbm, nch_hbm, zeros_hbm,
                     acc_out_hbm, sblk, dblk, rows, cvec,
                     isems, gsems, ssems, acc):
    cid = lax.axis_index("c")
    sid = lax.axis_index("s")
    wid = sid * _NC + cid
    r0 = sid * _RPT
    pltpu.sync_copy(zeros_hbm.at[pl.ds(r0, _RPT)], acc.at[pl.ds(r0, _RPT)])
    pltpu.sync_copy(nch_hbm.at[wid], cvec)
    plsc.subcore_barrier()

    nblk = jnp.max(cvec[...])

    def block(b, carry):
        o = pl.multiple_of(b * _BLK * _CH, 8)
        i0 = pltpu.async_copy(csrc_hbm.at[wid, pl.ds(o, _BLK * _CH)], sblk,
                              isems[0])
        i1 = pltpu.async_copy(cdst_hbm.at[wid, pl.ds(o, _BLK * _CH)], dblk,
                              isems[1])
        i0.wait(); i1.wait()
        g0 = pltpu.async_copy(xz_hbm.at[sblk.at[pl.ds(0, _CH)]], rows[0],
                              gsems[0])
        g1 = pltpu.async_copy(xz_hbm.at[sblk.at[pl.ds(_CH, _CH)]], rows[1],
                              gsems[1])
        g0.wait()
        s0 = pltpu.async_copy(rows[0], acc.at[dblk.at[pl.ds(0, _CH)]], ssems[0], add=True)
        g1.wait()
        s1 = pltpu.async_copy(rows[1], acc.at[dblk.at[pl.ds(_CH, _CH)]], ssems[1], add=True)
        s0.wait()
        g2 = pltpu.async_copy(xz_hbm.at[sblk.at[pl.ds(2 * _CH, _CH)]], rows[0],
                              gsems[0])
        s1.wait()
        g3 = pltpu.async_copy(xz_hbm.at[sblk.at[pl.ds(3 * _CH, _CH)]], rows[1],
                              gsems[1])
        g2.wait()
        s2 = pltpu.async_copy(rows[0], acc.at[dblk.at[pl.ds(2 * _CH, _CH)]], ssems[0], add=True)
        g3.wait()
        s3 = pltpu.async_copy(rows[1], acc.at[dblk.at[pl.ds(3 * _CH, _CH)]], ssems[1], add=True)
        s2.wait(); s3.wait()
        return carry

    lax.fori_loop(0, nblk, block, 0)

    plsc.subcore_barrier()
    pltpu.sync_copy(acc.at[pl.ds(r0, _RPT)], acc_out_hbm.at[cid, pl.ds(r0, _RPT)])


def _combine_body(acc_ref, cnt_ref, x_ref, f_ref, agg_ref, wn_ref, b_ref,
                  wr_ref, flag_ref, out_ref):
    summed = acc_ref[0] + acc_ref[1]
    count = cnt_ref[0] + cnt_ref[1]
    mean = summed / jnp.maximum(count, 1.0)
    f = f_ref[...]
    xz = x_ref[...] * (1.0 - f)
    agg = agg_ref[...]
    use_hybrid = flag_ref[0, 0] > 0.0
    target = (jnp.sum(jnp.abs(agg), axis=1, keepdims=True) > 0.0) & use_hybrid
    neigh_in = jnp.where(target, agg, mean)
    root_in = jnp.where(target, 0.0, xz)
    out_ref[...] = (
        jnp.dot(neigh_in, wn_ref[...], preferred_element_type=jnp.float32)
        + b_ref[...]
        + jnp.dot(root_in, wr_ref[...], preferred_element_type=jnp.float32))


def kernel(x, edge_index, frontier_mask, aggregated_neighbors,
           W_neigh, b_neigh, W_root):
    f = frontier_mask.astype(jnp.float32).reshape(_N, 1)
    fr_i = frontier_mask.astype(jnp.int32)
    src2d = jnp.pad(edge_index[0].reshape(_NCHK, _CH), ((0, 1), (0, 0)))
    dst2d = jnp.pad(edge_index[1].reshape(_NCHK, _CH), ((0, 1), (0, 0)))
    zeros = jnp.zeros((_NP, _D), jnp.float32)
    b2 = b_neigh.reshape(1, _D)

    xz, flag = pl.pallas_call(
        _build_xz_body,
        out_shape=[jax.ShapeDtypeStruct((_N, _D), jnp.float32),
                   jax.ShapeDtypeStruct((1, 1), jnp.float32)],
    )(x, f)

    mesh = plsc.VectorSubcoreMesh(core_axis_name="c", subcore_axis_name="s")
    params = pltpu.CompilerParams(use_tc_tiling_on_sc=False,
                                  needs_layout_passes=False)
    sc_compact = pl.kernel(
        _sc_compact_body,
        mesh=mesh,
        compiler_params=params,
        out_type=[jax.ShapeDtypeStruct((_NWK, _PCAP), jnp.int32),
                  jax.ShapeDtypeStruct((_NWK, _PCAP), jnp.int32),
                  jax.ShapeDtypeStruct((_NWK, 16), jnp.int32),
                  jax.ShapeDtypeStruct((_NC, _CROWS, 16), jnp.float32)],
        scratch_types=[
            pltpu.VMEM((_N,), jnp.int32),
            pltpu.VMEM((_CMAX, _CH), jnp.int32),
            pltpu.VMEM((_CMAX, _CH), jnp.int32),
            pltpu.VMEM((_PCAP,), jnp.int32),
            pltpu.VMEM((_PCAP,), jnp.int32),
            pltpu.VMEM((_CROWS, 16), jnp.float32),
            pltpu.VMEM((_CROWS // _CH, _CH), jnp.int32),
            pltpu.VMEM((16,), jnp.int32),
            pltpu.VMEM_SHARED((_CROWS, 16), jnp.float32),
        ],
    )
    iota_rows = jnp.arange(_CROWS, dtype=jnp.int32).reshape(_CROWS // _CH, _CH)
    csrc, cdst, nch, cnt2 = sc_compact(src2d, dst2d, fr_i, iota_rows)

    sc_scatter = pl.kernel(
        _sc_scatter_body,
        mesh=mesh,
        compiler_params=params,
        out_type=jax.ShapeDtypeStruct((_NC, _NP, _D), jnp.float32),
        scratch_types=[
            pltpu.VMEM((_BLK * _CH,), jnp.int32),
            pltpu.VMEM((_BLK * _CH,), jnp.int32),
            [pltpu.VMEM((_CH, _D), jnp.float32) for _ in range(2)],
            pltpu.VMEM((16,), jnp.int32),
            [pltpu.SemaphoreType.DMA for _ in range(2)],
            [pltpu.SemaphoreType.DMA for _ in range(2)],
            [pltpu.SemaphoreType.DMA for _ in range(2)],
            pltpu.VMEM_SHARED((_NP, _D), jnp.float32),
        ],
    )
    acc = sc_scatter(xz, csrc, cdst, nch, zeros)

    cnt3 = cnt2.reshape(_NC, _NP, 1)
    out = pl.pallas_call(
        _combine_body,
        grid=(_N // _BR,),
        in_specs=[
            pl.BlockSpec((_NC, _BR, _D), lambda i: (0, i, 0)),
            pl.BlockSpec((_NC, _BR, 1), lambda i: (0, i, 0)),
            pl.BlockSpec((_BR, _D), lambda i: (i, 0)),
            pl.BlockSpec((_BR, 1), lambda i: (i, 0)),
            pl.BlockSpec((_BR, _D), lambda i: (i, 0)),
            pl.BlockSpec((_D, _D), lambda i: (0, 0)),
            pl.BlockSpec((1, _D), lambda i: (0, 0)),
            pl.BlockSpec((_D, _D), lambda i: (0, 0)),
            pl.BlockSpec((1, 1), lambda i: (0, 0)),
        ],
        out_specs=pl.BlockSpec((_BR, _D), lambda i: (i, 0)),
        out_shape=jax.ShapeDtypeStruct((_N, _D), jnp.float32),
    )(acc, cnt3, x, f, aggregated_neighbors, W_neigh, b2, W_root, flag)
    return out


# speedup vs baseline: 1.9833x; 1.8123x over previous
import jax
import jax.numpy as jnp
from jax import lax
from jax.experimental import pallas as pl
from jax.experimental.pallas import tpu as pltpu
from jax.experimental.pallas import tpu_sc as plsc

_N = 10000
_E = 320000
_D = 128

_NC = 2
_NS = 16
_NWK = _NC * _NS
_CH = 128
_NCHK = _E // _CH
_CREM = _NCHK % _NWK
_CBASE = _NCHK // _NWK
_CMAX = _CBASE + 1
_BLK = 4
_PCAP = 80 * _CH
_NP = 10240
_RPT = _NP // _NS
_BR = 2000
_DUMP = _N + 64
_CROWS = _NP // 16


def _build_xz_body(x_ref, f_ref, xz_ref, flag_ref):
    f = f_ref[...]
    xz_ref[...] = x_ref[...] * (1.0 - f)
    flag_ref[...] = jnp.max(f).reshape(1, 1)


def _sc_compact_body(src2d_hbm, dst2d_hbm, fr_hbm, iota_hbm, csrc_hbm,
                     cdst_hbm, nch_hbm, cnt_out_hbm, fbuf, sall, dall, osrc,
                     odst, cbuf, iotab, cvec, cnt2):
    cid = lax.axis_index("c")
    sid = lax.axis_index("s")
    wid = sid * _NC + cid
    c0 = wid * _CBASE + lax.min(wid, _CREM)
    cnt = _CBASE + jnp.where(wid < _CREM, 1, 0)
    r0c = sid * (_CROWS // _NS)

    pltpu.sync_copy(fr_hbm, fbuf)
    pltpu.sync_copy(iota_hbm, iotab)
    pltpu.sync_copy(src2d_hbm.at[pl.ds(c0, _CMAX)], sall)
    pltpu.sync_copy(dst2d_hbm.at[pl.ds(c0, _CMAX)], dall)

    zsv = jnp.zeros((16,), jnp.float32)

    def zero_cbuf(i, carry):
        cbuf[i] = zsv
        return carry

    lax.fori_loop(0, _CROWS, zero_cbuf, 0)
    pltpu.sync_copy(cbuf.at[pl.ds(0, _CROWS // _NS)],
                    cnt2.at[pl.ds(r0c, _CROWS // _NS)])

    lane = lax.iota(jnp.int32, 16)

    def prefill(i, carry):
        spread = (lane + i * 16) & 127
        osrc[pl.ds(i * 16, 16)] = spread
        odst[pl.ds(i * 16, 16)] = _N + spread
        return carry

    lax.fori_loop(0, _PCAP // 16, prefill, 0)
    plsc.subcore_barrier()

    onev = jnp.ones((16,), jnp.float32)

    def chunk(j, cur):
        for t in range(_CH // 16):
            sv = sall[j, pl.ds(t * 16, 16)]
            dv = dall[j, pl.ds(t * 16, 16)]
            plsc.addupdate_scatter(cbuf, [dv >> 4, dv & 15], onev)
            fv = plsc.load_gather(fbuf, [sv])
            m = fv == 0
            plsc.store_compressed(osrc.at[pl.ds(cur, 16)], sv, mask=m)
            plsc.store_compressed(odst.at[pl.ds(cur, 16)], dv, mask=m)
            cur = cur + jnp.max(plsc.all_reduce_population_count(m))
        return cur

    cur = lax.fori_loop(0, cnt, chunk, jnp.int32(0))
    nblk = (cur + _BLK * _CH - 1) // (_BLK * _CH)

    for k in range(_CROWS // _CH):
        pltpu.sync_copy(cbuf.at[pl.ds(k * _CH, _CH)], cnt2.at[iotab.at[k]],
                        add=True)

    pltpu.sync_copy(osrc, csrc_hbm.at[wid])
    pltpu.sync_copy(odst, cdst_hbm.at[wid])
    cvec[...] = jnp.broadcast_to(nblk, (16,))
    pltpu.sync_copy(cvec, nch_hbm.at[wid])

    plsc.subcore_barrier()
    pltpu.sync_copy(cnt2.at[pl.ds(r0c, _CROWS // _NS)],
                    cnt_out_hbm.at[cid, pl.ds(r0c, _CROWS // _NS)])


def _sc_scatter_body(xz_hbm, csrc_hbm, cdst_hbm, nch_hbm, zeros_hbm,
                     acc_out_hbm, sblks, dblks, rows, cvec,
                     isems, gsems, ssems, acc):
    cid = lax.axis_index("c")
    sid = lax.axis_index("s")
    wid = sid * _NC + cid
    r0 = sid * _RPT
    pltpu.sync_copy(zeros_hbm.at[pl.ds(r0, _RPT)], acc.at[pl.ds(r0, _RPT)])
    pltpu.sync_copy(nch_hbm.at[wid], cvec)
    plsc.subcore_barrier()

    nblk = jnp.max(cvec[...])

    def fetch(b, p):
        o = pl.multiple_of(b * _BLK * _CH, 8)
        pltpu.async_copy(csrc_hbm.at[wid, pl.ds(o, _BLK * _CH)], sblks[p],
                         isems[2 * p])
        pltpu.async_copy(cdst_hbm.at[wid, pl.ds(o, _BLK * _CH)], dblks[p],
                         isems[2 * p + 1])

    def fwait(p):
        pltpu.make_async_copy(csrc_hbm.at[wid, pl.ds(0, _BLK * _CH)],
                              sblks[p], isems[2 * p]).wait()
        pltpu.make_async_copy(cdst_hbm.at[wid, pl.ds(0, _BLK * _CH)],
                              dblks[p], isems[2 * p + 1]).wait()

    def process(p):
        sblk, dblk = sblks[p], dblks[p]
        g0 = pltpu.async_copy(xz_hbm.at[sblk.at[pl.ds(0, _CH)]], rows[0],
                              gsems[0])
        g1 = pltpu.async_copy(xz_hbm.at[sblk.at[pl.ds(_CH, _CH)]], rows[1],
                              gsems[1])
        g0.wait()
        s0 = pltpu.async_copy(rows[0], acc.at[dblk.at[pl.ds(0, _CH)]],
                              ssems[0], add=True)
        g1.wait()
        s1 = pltpu.async_copy(rows[1], acc.at[dblk.at[pl.ds(_CH, _CH)]],
                              ssems[1], add=True)
        s0.wait()
        g2 = pltpu.async_copy(xz_hbm.at[sblk.at[pl.ds(2 * _CH, _CH)]], rows[0],
                              gsems[0])
        s1.wait()
        g3 = pltpu.async_copy(xz_hbm.at[sblk.at[pl.ds(3 * _CH, _CH)]], rows[1],
                              gsems[1])
        g2.wait()
        s2 = pltpu.async_copy(rows[0], acc.at[dblk.at[pl.ds(2 * _CH, _CH)]],
                              ssems[0], add=True)
        g3.wait()
        s3 = pltpu.async_copy(rows[1], acc.at[dblk.at[pl.ds(3 * _CH, _CH)]],
                              ssems[1], add=True)
        s2.wait(); s3.wait()

    @pl.when(nblk > 0)
    def _():
        fetch(0, 0)

        def pair(i, carry):
            b0 = 2 * i
            b1 = b0 + 1
            fwait(0)
            fetch(lax.min(b1, nblk - 1), 1)
            process(0)
            fwait(1)
            fetch(lax.min(b0 + 2, nblk - 1), 0)

            @pl.when(b1 < nblk)
            def _():
                process(1)

            return carry

        lax.fori_loop(0, (nblk + 1) // 2, pair, 0)
        fwait(0)

    plsc.subcore_barrier()
    pltpu.sync_copy(acc.at[pl.ds(r0, _RPT)], acc_out_hbm.at[cid, pl.ds(r0, _RPT)])


def _combine_body(acc_ref, cnt_ref, x_ref, f_ref, agg_ref, wn_ref, b_ref,
                  wr_ref, flag_ref, out_ref):
    summed = acc_ref[0] + acc_ref[1]
    count = cnt_ref[0] + cnt_ref[1]
    mean = summed / jnp.maximum(count, 1.0)
    f = f_ref[...]
    xz = x_ref[...] * (1.0 - f)
    agg = agg_ref[...]
    use_hybrid = flag_ref[0, 0] > 0.0
    target = (jnp.sum(jnp.abs(agg), axis=1, keepdims=True) > 0.0) & use_hybrid
    neigh_in = jnp.where(target, agg, mean)
    root_in = jnp.where(target, 0.0, xz)
    out_ref[...] = (
        jnp.dot(neigh_in, wn_ref[...], preferred_element_type=jnp.float32)
        + b_ref[...]
        + jnp.dot(root_in, wr_ref[...], preferred_element_type=jnp.float32))


def kernel(x, edge_index, frontier_mask, aggregated_neighbors,
           W_neigh, b_neigh, W_root):
    f = frontier_mask.astype(jnp.float32).reshape(_N, 1)
    fr_i = frontier_mask.astype(jnp.int32)
    src2d = jnp.pad(edge_index[0].reshape(_NCHK, _CH), ((0, 1), (0, 0)))
    dst2d = jnp.pad(edge_index[1].reshape(_NCHK, _CH), ((0, 1), (0, 0)))
    zeros = jnp.zeros((_NP, _D), jnp.float32)
    b2 = b_neigh.reshape(1, _D)

    xz, flag = pl.pallas_call(
        _build_xz_body,
        out_shape=[jax.ShapeDtypeStruct((_N, _D), jnp.float32),
                   jax.ShapeDtypeStruct((1, 1), jnp.float32)],
    )(x, f)

    mesh = plsc.VectorSubcoreMesh(core_axis_name="c", subcore_axis_name="s")
    params = pltpu.CompilerParams(use_tc_tiling_on_sc=False,
                                  needs_layout_passes=False)
    sc_compact = pl.kernel(
        _sc_compact_body,
        mesh=mesh,
        compiler_params=params,
        out_type=[jax.ShapeDtypeStruct((_NWK, _PCAP), jnp.int32),
                  jax.ShapeDtypeStruct((_NWK, _PCAP), jnp.int32),
                  jax.ShapeDtypeStruct((_NWK, 16), jnp.int32),
                  jax.ShapeDtypeStruct((_NC, _CROWS, 16), jnp.float32)],
        scratch_types=[
            pltpu.VMEM((_N,), jnp.int32),
            pltpu.VMEM((_CMAX, _CH), jnp.int32),
            pltpu.VMEM((_CMAX, _CH), jnp.int32),
            pltpu.VMEM((_PCAP,), jnp.int32),
            pltpu.VMEM((_PCAP,), jnp.int32),
            pltpu.VMEM((_CROWS, 16), jnp.float32),
            pltpu.VMEM((_CROWS // _CH, _CH), jnp.int32),
            pltpu.VMEM((16,), jnp.int32),
            pltpu.VMEM_SHARED((_CROWS, 16), jnp.float32),
        ],
    )
    iota_rows = jnp.arange(_CROWS, dtype=jnp.int32).reshape(_CROWS // _CH, _CH)
    csrc, cdst, nch, cnt2 = sc_compact(src2d, dst2d, fr_i, iota_rows)

    sc_scatter = pl.kernel(
        _sc_scatter_body,
        mesh=mesh,
        compiler_params=params,
        out_type=jax.ShapeDtypeStruct((_NC, _NP, _D), jnp.float32),
        scratch_types=[
            [pltpu.VMEM((_BLK * _CH,), jnp.int32) for _ in range(2)],
            [pltpu.VMEM((_BLK * _CH,), jnp.int32) for _ in range(2)],
            [pltpu.VMEM((_CH, _D), jnp.float32) for _ in range(2)],
            pltpu.VMEM((16,), jnp.int32),
            [pltpu.SemaphoreType.DMA for _ in range(4)],
            [pltpu.SemaphoreType.DMA for _ in range(2)],
            [pltpu.SemaphoreType.DMA for _ in range(2)],
            pltpu.VMEM_SHARED((_NP, _D), jnp.float32),
        ],
    )
    acc = sc_scatter(xz, csrc, cdst, nch, zeros)

    cnt3 = cnt2.reshape(_NC, _NP, 1)
    out = pl.pallas_call(
        _combine_body,
        grid=(_N // _BR,),
        in_specs=[
            pl.BlockSpec((_NC, _BR, _D), lambda i: (0, i, 0)),
            pl.BlockSpec((_NC, _BR, 1), lambda i: (0, i, 0)),
            pl.BlockSpec((_BR, _D), lambda i: (i, 0)),
            pl.BlockSpec((_BR, 1), lambda i: (i, 0)),
            pl.BlockSpec((_BR, _D), lambda i: (i, 0)),
            pl.BlockSpec((_D, _D), lambda i: (0, 0)),
            pl.BlockSpec((1, _D), lambda i: (0, 0)),
            pl.BlockSpec((_D, _D), lambda i: (0, 0)),
            pl.BlockSpec((1, 1), lambda i: (0, 0)),
        ],
        out_specs=pl.BlockSpec((_BR, _D), lambda i: (i, 0)),
        out_shape=jax.ShapeDtypeStruct((_N, _D), jnp.float32),
    )(acc, cnt3, x, f, aggregated_neighbors, W_neigh, b2, W_root, flag)
    return out
